# Initial kernel scaffold; baseline (speedup 1.0000x reference)
import jax
import jax.numpy as jnp
from jax.experimental import pallas as pl


def kernel(ents_w, rels_w, heads, rels, tails, sources, heads_bad, rels_bad, tails_bad, sources_bad):
    raise NotImplementedError("write your pallas kernel here")


# speedup vs baseline: 1.1500x; 1.1500x over previous
import functools

import jax
import jax.numpy as jnp
from jax import lax
from jax.experimental import pallas as pl
from jax.experimental.pallas import tpu as pltpu
from jax.experimental.pallas import tpu_sc as plsc

DIM = 64
LANES = 16
NC = 2
NS = 16
NW = NC * NS
TOTAL = 32768
PER_W = TOTAL // NW
CH = 512
NCHUNK = PER_W // CH
GSL = 128
K = CH // GSL

EPS2 = 1e-24
TINY = 1e-35


def _fast_rsqrt(x):
    i = lax.bitcast_convert_type(x, jnp.int32)
    i = jnp.int32(0x5F3759DF) - lax.shift_right_arithmetic(i, 1)
    y = lax.bitcast_convert_type(i, jnp.float32)
    half_x = 0.5 * x
    for _ in range(3):
        y = y * (1.5 - half_x * y * y)
    return y


def _transe_body(ents_hbm, rels_hbm, hidx_hbm, ridx_hbm, tidx_hbm, out_hbm,
                 hidx_v, ridx_v, tidx_v, hrows, rrows, trows, scores_v, sem):
    wid = lax.axis_index("s") * NC + lax.axis_index("c")
    iota = lax.iota(jnp.int32, LANES)

    for c in range(NCHUNK):
        base = wid * PER_W + c * CH
        pltpu.sync_copy(hidx_hbm.at[pl.ds(base, CH)], hidx_v)
        pltpu.sync_copy(ridx_hbm.at[pl.ds(base, CH)], ridx_v)
        pltpu.sync_copy(tidx_hbm.at[pl.ds(base, CH)], tidx_v)
        copies = []
        for k in range(K):
            sl = pl.ds(k * GSL, GSL)
            copies.append(pltpu.async_copy(ents_hbm.at[hidx_v.at[sl]], hrows.at[sl], sem))
            copies.append(pltpu.async_copy(rels_hbm.at[ridx_v.at[sl]], rrows.at[sl], sem))
            copies.append(pltpu.async_copy(ents_hbm.at[tidx_v.at[sl]], trows.at[sl], sem))
        for cp in copies:
            cp.wait()

        def o_body(oi, _):
            rows16 = oi * LANES + iota
            z = jnp.zeros((LANES,), jnp.float32)
            s_h, s_r, s_t, d_hr, d_ht, d_rt = z, z, z, z, z, z
            for d in range(DIM):
                dcol = jnp.full((LANES,), d, jnp.int32)
                vh = plsc.load_gather(hrows, [rows16, dcol])
                vr = plsc.load_gather(rrows, [rows16, dcol])
                vt = plsc.load_gather(trows, [rows16, dcol])
                s_h = s_h + vh * vh
                s_r = s_r + vr * vr
                s_t = s_t + vt * vt
                d_hr = d_hr + vh * vr
                d_ht = d_ht + vh * vt
                d_rt = d_rt + vr * vt
            ih = _fast_rsqrt(jnp.maximum(s_h, EPS2))
            ir = _fast_rsqrt(jnp.maximum(s_r, EPS2))
            it = _fast_rsqrt(jnp.maximum(s_t, EPS2))
            ssq = (s_h * ih * ih + s_r * ir * ir + s_t * it * it
                   + 2.0 * (d_hr * (ih * ir) - d_ht * (ih * it) - d_rt * (ir * it)))
            ssq = jnp.maximum(ssq, 0.0)
            scores_v[pl.ds(oi * LANES, LANES)] = ssq * _fast_rsqrt(jnp.maximum(ssq, TINY))
            return 0

        lax.fori_loop(0, CH // LANES, o_body, 0)
        pltpu.sync_copy(scores_v, out_hbm.at[pl.ds(base, CH)])


@functools.partial(
    pl.kernel,
    out_type=jax.ShapeDtypeStruct((TOTAL,), jnp.float32),
    mesh=plsc.VectorSubcoreMesh(core_axis_name="c", subcore_axis_name="s"),
    compiler_params=pltpu.CompilerParams(
        needs_layout_passes=False, use_tc_tiling_on_sc=False),
    scratch_types=[
        pltpu.VMEM((CH,), jnp.int32),
        pltpu.VMEM((CH,), jnp.int32),
        pltpu.VMEM((CH,), jnp.int32),
        pltpu.VMEM((CH, DIM), jnp.float32),
        pltpu.VMEM((CH, DIM), jnp.float32),
        pltpu.VMEM((CH, DIM), jnp.float32),
        pltpu.VMEM((CH,), jnp.float32),
        pltpu.SemaphoreType.DMA,
    ],
)
def _transe_sc(*refs):
    _transe_body(*refs)


def kernel(ents_w, rels_w, heads, rels, tails, sources, heads_bad, rels_bad,
           tails_bad, sources_bad):
    b = heads.shape[0]
    all_heads = jnp.concatenate([heads, heads_bad])
    all_rels = jnp.concatenate([rels, rels_bad])
    all_tails = jnp.concatenate([tails, tails_bad])
    out = _transe_sc(ents_w, rels_w, all_heads, all_rels, all_tails)
    return out[:b], out[b:]
